# baseline (speedup 1.0000x reference)
import jax
import jax.numpy as jnp
from jax import lax
from jax.experimental import pallas as pl
from jax.experimental.pallas import tpu as pltpu

_R = 2
_EPS = 1e-5


def _fused_kernel(x_ref, m2_ref, p_ref, r0_ref, r1_ref, o_ref):
    C, S = x_ref.shape
    _, D, _, Hr, Wr = o_ref.shape
    H, W = Hr // _R, Wr // _R
    r3 = _R * _R * _R
    total = float(S * r3 * C)
    sr3 = float(S * r3)

    x = x_ref[...]
    sx = jnp.sum(x, axis=1, keepdims=True)
    sxx = jnp.sum(x * x, axis=1, keepdims=True)

    a_c = p_ref[:, 0:1]
    b_c = p_ref[:, 1:2]
    bias = p_ref[:, 2:3]
    p1 = p_ref[:, 3:4]
    p2 = p_ref[:, 4:5]
    p3 = p_ref[:, 5:6]

    s1 = jnp.sum(sx * a_c) + sr3 * jnp.sum(bias)
    mean = s1 / total
    d = bias - mean
    s2 = (jnp.sum(sxx * b_c)
          + 2.0 * jnp.sum(sx * a_c * d)
          + sr3 * jnp.sum(d * d))
    inv_std = lax.rsqrt(s2 / total + _EPS)

    y = jnp.dot(m2_ref[0], x * inv_std,
                preferred_element_type=jnp.float32)

    const_c = inv_std * (p1 - mean * p2) + p3

    rk = (r0_ref[...], r1_ref[...])
    dims = (((2,), (0,)), ((), ()))
    for j in range(_R):
        zj = None
        for k in range(_R):
            yjk = y[(j * _R + k) * C:(j * _R + k + 1) * C, :]
            y3 = yjk.reshape(C, D * H, W)
            part = lax.dot_general(y3, rk[k], dims,
                                   preferred_element_type=jnp.float32)
            zj = part if zj is None else zj + part
        zj = zj + const_c[:, :, None]
        zj = zj.reshape(C, D, H, Wr)
        o_ref[:, :, 0, pl.Slice(j, H, _R), :] = zj


def kernel(x, conv_t_w, conv_t_b, gn_w, gn_b, pw_w):
    N, C, D, H, W = x.shape
    r = _R
    r3 = r * r * r
    S = D * H * W
    f32 = jnp.float32

    xf = x.reshape(N, C, S).astype(f32)
    wt = conv_t_w.reshape(C, r3).astype(f32)
    bias = conv_t_b.astype(f32)
    gamma = gn_w.astype(f32)
    beta = gn_b.astype(f32)
    wpw = pw_w.reshape(C, C).T.astype(f32)

    m2 = (wt.T[:, None, :] * gamma[None, None, :]
          * wpw.T[None, :, :]).reshape(r, 4 * C, C)

    a_vec = jnp.sum(wt, axis=1)
    b_vec = jnp.sum(wt * wt, axis=1)
    p1 = (bias * gamma) @ wpw
    p2 = gamma @ wpw
    p3 = beta @ wpw
    zero = jnp.zeros((C,), f32)
    p_cols = jnp.stack([a_vec, b_vec, bias, p1, p2, p3, zero, zero], axis=1)

    wi = jnp.arange(W)[:, None]
    li = jnp.arange(r * W)[None, :]
    r0 = (li == r * wi).astype(f32)
    r1 = (li == r * wi + 1).astype(f32)

    out = pl.pallas_call(
        _fused_kernel,
        out_shape=jax.ShapeDtypeStruct((N, C, D, r, H * r, W * r), f32),
        grid=(N, r),
        in_specs=[
            pl.BlockSpec((None, C, S), lambda n, i: (n, 0, 0)),
            pl.BlockSpec((1, 4 * C, C), lambda n, i: (i, 0, 0)),
            pl.BlockSpec((C, 8), lambda n, i: (0, 0)),
            pl.BlockSpec((W, r * W), lambda n, i: (0, 0)),
            pl.BlockSpec((W, r * W), lambda n, i: (0, 0)),
        ],
        out_specs=pl.BlockSpec((None, C, D, 1, H * r, W * r),
                               lambda n, i: (n, 0, 0, i, 0, 0)),
        compiler_params=pltpu.CompilerParams(
            dimension_semantics=("parallel", "arbitrary")),
    )(xf, m2, p_cols, r0, r1)

    return out.reshape(N, C, D * r, H * r, W * r).astype(x.dtype)


# speedup vs baseline: 2.4524x; 2.4524x over previous
import jax
import jax.numpy as jnp
from jax import lax
from jax.experimental import pallas as pl
from jax.experimental.pallas import tpu as pltpu

_R = 2
_EPS = 1e-5


def _fused_kernel(x_ref, m2_ref, p_ref, rp_ref, o_ref):
    C, rows, grp = x_ref.shape
    _, D, _, SUB, LANE = o_ref.shape
    S = rows * grp
    r3 = _R * _R * _R
    total = float(S * r3 * C)
    sr3 = float(S * r3)

    x = x_ref[...]
    sx = jnp.sum(x, axis=(1, 2)).reshape(C, 1)
    sxx = jnp.sum(x * x, axis=(1, 2)).reshape(C, 1)

    a_c = p_ref[:, 0:1]
    b_c = p_ref[:, 1:2]
    bias = p_ref[:, 2:3]
    p1 = p_ref[:, 3:4]
    p2 = p_ref[:, 4:5]
    p3 = p_ref[:, 5:6]

    s1 = jnp.sum(sx * a_c) + sr3 * jnp.sum(bias)
    mean = s1 / total
    d = bias - mean
    s2 = (jnp.sum(sxx * b_c)
          + 2.0 * jnp.sum(sx * a_c * d)
          + sr3 * jnp.sum(d * d))
    inv_std = lax.rsqrt(s2 / total + _EPS)

    xs = x * inv_std
    sdims = (((2,), (0,)), ((), ()))
    xg = [
        lax.dot_general(xs, rp_ref[g], sdims,
                        preferred_element_type=jnp.float32)
        for g in range(4)
    ]
    xgc = jnp.concatenate(xg, axis=0)

    mdims = (((1,), (0,)), ((), ()))
    z = lax.dot_general(m2_ref[0], xgc, mdims,
                        preferred_element_type=jnp.float32)

    const_c = inv_std * (p1 - mean * p2) + p3
    z = z + const_c[:, :, None]
    o_ref[...] = z.reshape(C, D, 1, SUB, LANE)


def kernel(x, conv_t_w, conv_t_b, gn_w, gn_b, pw_w):
    N, C, D, H, W = x.shape
    r = _R
    r3 = r * r * r
    S = D * H * W
    f32 = jnp.float32

    grp = r * W
    xf = x.reshape(N, C, S // grp, grp).astype(f32)
    wt = conv_t_w.reshape(C, r3).astype(f32)
    bias = conv_t_b.astype(f32)
    gamma = gn_w.astype(f32)
    beta = gn_b.astype(f32)
    wpw = pw_w.reshape(C, C).T.astype(f32)

    m2 = (wt.T[:, None, :] * gamma[None, None, :]
          * wpw.T[None, :, :]).reshape(r, 4, C, C)
    mcat = jnp.transpose(m2, (0, 2, 1, 3)).reshape(r, C, 4 * C)

    a_vec = jnp.sum(wt, axis=1)
    b_vec = jnp.sum(wt * wt, axis=1)
    p1 = (bias * gamma) @ wpw
    p2 = gamma @ wpw
    p3 = beta @ wpw
    zero = jnp.zeros((C,), f32)
    p_cols = jnp.stack([a_vec, b_vec, bias, p1, p2, p3, zero, zero], axis=1)

    jj, kk, hh, ww = jnp.meshgrid(jnp.arange(r), jnp.arange(r),
                                  jnp.arange(r), jnp.arange(W),
                                  indexing="ij")
    g_idx = (jj * r + kk).ravel()
    src = (hh * W + ww).ravel()
    dst = (((hh * r + jj) * W + ww) * r + kk).ravel()
    rp = jnp.zeros((4, r * W, 4 * r * W), f32)
    rp = rp.at[g_idx, src, dst].set(1.0)

    out = pl.pallas_call(
        _fused_kernel,
        out_shape=jax.ShapeDtypeStruct((N, C, D, r, H * r // 4, 4 * r * W),
                                       f32),
        grid=(N, r),
        in_specs=[
            pl.BlockSpec((None, C, S // grp, grp), lambda n, i: (n, 0, 0, 0)),
            pl.BlockSpec((1, C, 4 * C), lambda n, i: (i, 0, 0)),
            pl.BlockSpec((C, 8), lambda n, i: (0, 0)),
            pl.BlockSpec((4, r * W, 4 * r * W), lambda n, i: (0, 0, 0)),
        ],
        out_specs=pl.BlockSpec((None, C, D, 1, H * r // 4, 4 * r * W),
                               lambda n, i: (n, 0, 0, i, 0, 0)),
        compiler_params=pltpu.CompilerParams(
            dimension_semantics=("parallel", "arbitrary")),
    )(xf, mcat, p_cols, rp)

    return out.reshape(N, C, D * r, H * r, W * r).astype(x.dtype)


# speedup vs baseline: 3.2770x; 1.3362x over previous
import jax
import jax.numpy as jnp
from jax import lax
from jax.experimental import pallas as pl
from jax.experimental.pallas import tpu as pltpu

_R = 2
_EPS = 1e-5


def _fused_kernel(x_ref, m2_ref, p_ref, rp_ref, o_ref):
    C, rows, grp = x_ref.shape
    _, D, _, SUB, LANE = o_ref.shape
    S = rows * grp
    r3 = _R * _R * _R
    total = float(S * r3 * C)
    sr3 = float(S * r3)

    x = x_ref[...]
    sx = jnp.sum(x, axis=(1, 2)).reshape(C, 1)
    sxx = jnp.sum(x * x, axis=(1, 2)).reshape(C, 1)

    a_c = p_ref[:, 0:1]
    b_c = p_ref[:, 1:2]
    bias = p_ref[:, 2:3]
    p1 = p_ref[:, 3:4]
    p2 = p_ref[:, 4:5]
    p3 = p_ref[:, 5:6]

    s1 = jnp.sum(sx * a_c) + sr3 * jnp.sum(bias)
    mean = s1 / total
    d = bias - mean
    s2 = (jnp.sum(sxx * b_c)
          + 2.0 * jnp.sum(sx * a_c * d)
          + sr3 * jnp.sum(d * d))
    inv_std = lax.rsqrt(s2 / total + _EPS)

    xs = x * inv_std
    sdims = (((2,), (0,)), ((), ()))
    xg = [
        lax.dot_general(xs, rp_ref[g], sdims,
                        preferred_element_type=jnp.float32)
        for g in range(4)
    ]
    xgc = jnp.concatenate(xg, axis=0)

    mdims = (((1,), (0,)), ((), ()))
    z = lax.dot_general(m2_ref[...], xgc, mdims,
                        preferred_element_type=jnp.float32)

    const_c = inv_std * (p1 - mean * p2) + p3
    const_2c = jnp.concatenate([const_c, const_c], axis=0)
    z = z + const_2c[:, :, None]
    z = z.reshape(_R, C, D, SUB, LANE)
    o_ref[...] = jnp.transpose(z, (1, 2, 0, 3, 4))


def kernel(x, conv_t_w, conv_t_b, gn_w, gn_b, pw_w):
    N, C, D, H, W = x.shape
    r = _R
    r3 = r * r * r
    S = D * H * W
    f32 = jnp.float32

    grp = r * W
    xf = x.reshape(N, C, S // grp, grp).astype(f32)
    wt = conv_t_w.reshape(C, r3).astype(f32)
    bias = conv_t_b.astype(f32)
    gamma = gn_w.astype(f32)
    beta = gn_b.astype(f32)
    wpw = pw_w.reshape(C, C).T.astype(f32)

    m2 = (wt.T[:, None, :] * gamma[None, None, :]
          * wpw.T[None, :, :]).reshape(r, 4, C, C)
    mcat = jnp.transpose(m2, (0, 2, 1, 3)).reshape(r * C, 4 * C)

    a_vec = jnp.sum(wt, axis=1)
    b_vec = jnp.sum(wt * wt, axis=1)
    p1 = (bias * gamma) @ wpw
    p2 = gamma @ wpw
    p3 = beta @ wpw
    zero = jnp.zeros((C,), f32)
    p_cols = jnp.stack([a_vec, b_vec, bias, p1, p2, p3, zero, zero], axis=1)

    jj, kk, hh, ww = jnp.meshgrid(jnp.arange(r), jnp.arange(r),
                                  jnp.arange(r), jnp.arange(W),
                                  indexing="ij")
    g_idx = (jj * r + kk).ravel()
    src = (hh * W + ww).ravel()
    dst = (((hh * r + jj) * W + ww) * r + kk).ravel()
    rp = jnp.zeros((4, r * W, 4 * r * W), f32)
    rp = rp.at[g_idx, src, dst].set(1.0)

    out = pl.pallas_call(
        _fused_kernel,
        out_shape=jax.ShapeDtypeStruct((N, C, D, r, H * r // 4, 4 * r * W),
                                       f32),
        grid=(N,),
        in_specs=[
            pl.BlockSpec((None, C, S // grp, grp), lambda n: (n, 0, 0, 0)),
            pl.BlockSpec((r * C, 4 * C), lambda n: (0, 0)),
            pl.BlockSpec((C, 8), lambda n: (0, 0)),
            pl.BlockSpec((4, r * W, 4 * r * W), lambda n: (0, 0, 0)),
        ],
        out_specs=pl.BlockSpec((None, C, D, r, H * r // 4, 4 * r * W),
                               lambda n: (n, 0, 0, 0, 0, 0)),
        compiler_params=pltpu.CompilerParams(
            dimension_semantics=("parallel",)),
    )(xf, mcat, p_cols, rp)

    return out.reshape(N, C, D * r, H * r, W * r).astype(x.dtype)


# speedup vs baseline: 3.7131x; 1.1331x over previous
import jax
import jax.numpy as jnp
from jax import lax
from jax.experimental import pallas as pl
from jax.experimental.pallas import tpu as pltpu

_R = 2
_EPS = 1e-5


def _fused_kernel(x_ref, t_ref, p_ref, c_ref, o_ref):
    rows, CL = x_ref.shape
    IO, _, LANE = t_ref.shape
    C, D, _, SUB, _ = o_ref.shape
    r3 = _R * _R * _R
    S = rows * CL // C
    total = float(S * r3 * C)
    sr3 = float(S * r3)

    x = x_ref[...]
    a512 = p_ref[0:1, :]
    ab512 = p_ref[1:2, :]
    b512 = p_ref[2:3, :]
    sumb = p_ref[3, 0]
    sumb2 = p_ref[3, 1]

    sax = jnp.sum(x * a512)
    sabx = jnp.sum(x * (a512 * ab512))
    sbxx = jnp.sum((x * x) * b512)

    s1 = sax + sr3 * sumb
    mean = s1 / total
    s2 = (sbxx
          + 2.0 * (sabx - mean * sax)
          + sr3 * (sumb2 - 2.0 * mean * sumb + C * mean * mean))
    inv_std = lax.rsqrt(s2 / total + _EPS)

    xs = x * inv_std

    cst = (inv_std * (c_ref[:, 0:1] - mean * c_ref[:, 1:2])
           + c_ref[:, 2:3])

    zs = [
        jnp.dot(xs, t_ref[io], preferred_element_type=jnp.float32)
        for io in range(IO)
    ]
    z = jnp.stack(zs, axis=0) + cst[:, :, None]
    z = z.reshape(_R, C, D, SUB, LANE)
    o_ref[...] = jnp.transpose(z, (1, 2, 0, 3, 4))


def kernel(x, conv_t_w, conv_t_b, gn_w, gn_b, pw_w):
    N, C, D, H, W = x.shape
    r = _R
    r3 = r * r * r
    S = D * H * W
    f32 = jnp.float32
    grp = r * W
    rows = S // grp

    xf = jnp.transpose(x.reshape(N, C, rows, grp).astype(f32),
                       (0, 2, 1, 3)).reshape(N, rows, C * grp)

    wt = conv_t_w.reshape(C, r3).astype(f32)
    bias = conv_t_b.astype(f32)
    gamma = gn_w.astype(f32)
    beta = gn_b.astype(f32)
    wpw = pw_w.reshape(C, C).T.astype(f32)

    jj, kk, hh, ww = jnp.meshgrid(jnp.arange(r), jnp.arange(r),
                                  jnp.arange(r), jnp.arange(W),
                                  indexing="ij")
    lane_of = (((hh * r + jj) * W + ww) * r + kk).ravel()
    src_of = (hh * W + ww).ravel()
    jk_of = (jj * r + kk).ravel()
    t_full = jnp.zeros((r, C, C, grp, 4 * r * W), f32)
    for i in range(r):
        wg = wt[:, i * 4 + jk_of] * gamma[:, None]
        vals = wg[:, None, :] * wpw[:, :, None]
        t_full = t_full.at[i, :, :, src_of, lane_of].add(
            jnp.transpose(vals, (2, 0, 1)))
    t_mat = jnp.transpose(t_full, (0, 2, 1, 3, 4)).reshape(
        r * C, C * grp, 4 * r * W)

    a_vec = jnp.sum(wt, axis=1)
    b_vec = jnp.sum(wt * wt, axis=1)
    ones_g = jnp.ones((1, grp), f32)
    a512 = (a_vec[:, None] * ones_g).reshape(1, C * grp)
    b512 = (b_vec[:, None] * ones_g).reshape(1, C * grp)
    ab512 = (bias[:, None] * ones_g).reshape(1, C * grp)
    scal = jnp.zeros((1, C * grp), f32)
    scal = scal.at[0, 0].set(jnp.sum(bias))
    scal = scal.at[0, 1].set(jnp.sum(bias * bias))
    p_rows = jnp.concatenate([a512, ab512, b512, scal], axis=0)

    q1 = (bias * gamma) @ wpw
    q2 = gamma @ wpw
    q3 = beta @ wpw
    qs = jnp.stack([q1, q2, q3] + [jnp.zeros(C, f32)] * 5, axis=1)
    q2c = jnp.concatenate([qs, qs], axis=0)

    out = pl.pallas_call(
        _fused_kernel,
        out_shape=jax.ShapeDtypeStruct((N, C, D, r, H * r // 4, 4 * r * W),
                                       f32),
        grid=(N,),
        in_specs=[
            pl.BlockSpec((None, rows, C * grp), lambda n: (n, 0, 0)),
            pl.BlockSpec((r * C, C * grp, 4 * r * W), lambda n: (0, 0, 0)),
            pl.BlockSpec((4, C * grp), lambda n: (0, 0)),
            pl.BlockSpec((r * C, 8), lambda n: (0, 0)),
        ],
        out_specs=pl.BlockSpec((None, C, D, r, H * r // 4, 4 * r * W),
                               lambda n: (n, 0, 0, 0, 0, 0)),
        compiler_params=pltpu.CompilerParams(
            dimension_semantics=("parallel",)),
    )(xf, t_mat, p_rows, q2c)

    return out.reshape(N, C, D * r, H * r, W * r).astype(x.dtype)
